# Initial kernel scaffold; baseline (speedup 1.0000x reference)
import functools

import jax
import jax.numpy as jnp
from jax import lax
from jax.experimental import pallas as pl
from jax.experimental.pallas import tpu as pltpu
from jax.experimental.pallas import tpu_sc as plsc

NN = 10000
EE = 320000
DD = 128
LL = 4
NWGT = 20
NC, NS = 2, 16
NW = NC * NS
NPAD = 10240
RPT = NPAD // NW
RPS = NPAD // NS
EPT = EE // NW
ECH = 400
NCHUNK = EPT // ECH

_MESH = plsc.VectorSubcoreMesh(
    core_axis_name="c", subcore_axis_name="s", num_cores=NC, num_subcores=NS)


def _embed_body(table, idx, out, idx_v, rows_v, sem):
  wid = lax.axis_index("s") * NC + lax.axis_index("c")
  base = wid * RPT
  pltpu.sync_copy(idx.at[pl.ds(base, RPT)], idx_v)
  pltpu.async_copy(table.at[idx_v], rows_v, sem).wait()
  pltpu.sync_copy(rows_v, out.at[pl.ds(base, RPT)])


_embed_gather = pl.kernel(
    _embed_body,
    out_type=jax.ShapeDtypeStruct((NPAD, DD), jnp.float32),
    mesh=_MESH,
    scratch_types=[
        pltpu.VMEM((RPT,), jnp.int32),
        pltpu.VMEM((RPT, DD), jnp.float32),
        pltpu.SemaphoreType.DMA,
    ],
)


def _agg_body(tbl, src2, dst, zrows, part, acc_sh, src_v, dst_v, rows_v, sem):
  c = lax.axis_index("c")
  s = lax.axis_index("s")
  wid = s * NC + c
  rbase = s * RPS
  pltpu.sync_copy(zrows.at[pl.ds(rbase, RPS)], acc_sh.at[pl.ds(rbase, RPS)])
  plsc.subcore_barrier()

  ebase = wid * EPT

  def chunk(i, carry):
    b = ebase + i * ECH
    pltpu.sync_copy(src2.at[pl.ds(b, ECH)], src_v)
    pltpu.sync_copy(dst.at[pl.ds(b, ECH)], dst_v)
    pltpu.async_copy(tbl.at[src_v], rows_v, sem).wait()
    pltpu.sync_copy(rows_v, acc_sh.at[dst_v], add=True)
    return carry

  lax.fori_loop(0, NCHUNK, chunk, 0)
  plsc.subcore_barrier()
  pltpu.sync_copy(acc_sh.at[pl.ds(rbase, RPS)], part.at[c, pl.ds(rbase, RPS)])


_edge_aggregate = pl.kernel(
    _agg_body,
    out_type=jax.ShapeDtypeStruct((NC, NPAD, DD), jnp.float32),
    mesh=_MESH,
    scratch_types=[
        pltpu.VMEM_SHARED((NPAD, DD), jnp.float32),
        pltpu.VMEM((ECH,), jnp.int32),
        pltpu.VMEM((ECH,), jnp.int32),
        pltpu.VMEM((ECH, DD), jnp.float32),
        pltpu.SemaphoreType.DMA,
    ],
)


def _prep_body(ei0_ref, ea_ref, src2_ref):
  src2_ref[...] = ea_ref[...] * NN + ei0_ref[...]


def _prep_src2(ei0, ea):
  return pl.pallas_call(
      _prep_body,
      out_shape=jax.ShapeDtypeStruct((EE // DD, DD), jnp.int32),
  )(ei0.reshape(EE // DD, DD), ea.reshape(EE // DD, DD)).reshape(EE)


_MB = 400


def _mscale_body(h_ref, w_ref, tbl_ref):
  mb = jnp.dot(h_ref[...], w_ref[...], preferred_element_type=jnp.float32)
  for w in range(NWGT):
    tbl_ref[w] = mb * jnp.float32(w)


def _mscale(h, w):
  return pl.pallas_call(
      _mscale_body,
      grid=(NN // _MB,),
      in_specs=[
          pl.BlockSpec((_MB, DD), lambda i: (i, 0)),
          pl.BlockSpec((DD, DD), lambda i: (0, 0)),
      ],
      out_specs=pl.BlockSpec((NWGT, _MB, DD), lambda i: (0, i, 0)),
      out_shape=jax.ShapeDtypeStruct((NWGT, NN, DD), jnp.float32),
  )(h, w).reshape(NWGT * NN, DD)


_GB = 1024


def _gru_body(part_ref, h_ref, wih_ref, whh_ref, bih_ref, bhh_ref, out_ref):
  agg = part_ref[0] + part_ref[1]
  h = h_ref[...]
  gi = lax.dot_general(agg, wih_ref[...], (((1,), (1,)), ((), ())),
                       preferred_element_type=jnp.float32) + bih_ref[...]
  gh = lax.dot_general(h, whh_ref[...], (((1,), (1,)), ((), ())),
                       preferred_element_type=jnp.float32) + bhh_ref[...]
  i_r, i_z, i_n = gi[:, :DD], gi[:, DD:2 * DD], gi[:, 2 * DD:]
  h_r, h_z, h_n = gh[:, :DD], gh[:, DD:2 * DD], gh[:, 2 * DD:]
  r = jax.nn.sigmoid(i_r + h_r)
  z = jax.nn.sigmoid(i_z + h_z)
  n = jnp.tanh(i_n + r * h_n)
  out_ref[...] = (1.0 - z) * n + z * h


def _gru(part, h, w_ih, w_hh, b_ih, b_hh):
  return pl.pallas_call(
      _gru_body,
      grid=(NPAD // _GB,),
      in_specs=[
          pl.BlockSpec((NC, _GB, DD), lambda i: (0, i, 0)),
          pl.BlockSpec((_GB, DD), lambda i: (i, 0)),
          pl.BlockSpec((3 * DD, DD), lambda i: (0, 0)),
          pl.BlockSpec((3 * DD, DD), lambda i: (0, 0)),
          pl.BlockSpec((1, 3 * DD), lambda i: (0, 0)),
          pl.BlockSpec((1, 3 * DD), lambda i: (0, 0)),
      ],
      out_specs=pl.BlockSpec((_GB, DD), lambda i: (i, 0)),
      out_shape=jax.ShapeDtypeStruct((NPAD, DD), jnp.float32),
  )(part, h, w_ih, w_hh, b_ih.reshape(1, 3 * DD), b_hh.reshape(1, 3 * DD))


def _pool_body(h_ref, gw_ref, gb_ref, lw_ref, lb_ref, y_ref):
  h = h_ref[...]
  scores = jnp.sum(h * gw_ref[...], axis=1, keepdims=True) + gb_ref[0, 0]
  gate = jax.nn.sigmoid(scores)
  rid = lax.broadcasted_iota(jnp.int32, (NPAD, 1), 0)
  valid = rid < NN
  gate = jnp.where(valid, gate, -jnp.inf)
  gate = gate - jnp.max(gate, axis=0, keepdims=True)
  e = jnp.exp(gate)
  e = jnp.where(valid, e, 0.0)
  g = e / jnp.sum(e, axis=0, keepdims=True)
  hg = jnp.sum(g * h, axis=0, keepdims=True)
  y_ref[...] = lax.dot_general(hg, lw_ref[...], (((1,), (1,)), ((), ())),
                               preferred_element_type=jnp.float32) + lb_ref[...]


def _pool(h, gate_w, gate_b, label_w, label_b):
  return pl.pallas_call(
      _pool_body,
      out_shape=jax.ShapeDtypeStruct((1, 2), jnp.float32),
  )(h, gate_w, gate_b.reshape(1, 1), label_w, label_b.reshape(1, 2))


@jax.jit
def kernel(x, edge_index, edge_attr, embed_table, edge_embed_table,
           ggnn_weight, gru_w_ih, gru_w_hh, gru_b_ih, gru_b_hh,
           gate_w, gate_b, label_w, label_b):
  del edge_embed_table

  xpad = jnp.concatenate(
      [x[:, 0].astype(jnp.int32),
       jnp.zeros((NPAD - NN,), jnp.int32)])
  h = _embed_gather(embed_table, xpad)

  src2 = _prep_src2(edge_index[0].astype(jnp.int32),
                    edge_attr[:, 0].astype(jnp.int32))
  dst = edge_index[1].astype(jnp.int32)
  zrows = jnp.zeros((NPAD, DD), jnp.float32)

  for l in range(LL):
    tbl = _mscale(h, ggnn_weight[l])
    part = _edge_aggregate(tbl, src2, dst, zrows)
    h = _gru(part, h, gru_w_ih, gru_w_hh, gru_b_ih, gru_b_hh)

  return _pool(h, gate_w, gate_b, label_w, label_b)


# speedup vs baseline: 6.3757x; 6.3757x over previous
import functools

import jax
import jax.numpy as jnp
from jax import lax
from jax.experimental import pallas as pl
from jax.experimental.pallas import tpu as pltpu
from jax.experimental.pallas import tpu_sc as plsc

NN = 10000
EE = 320000
DD = 128
LL = 4
NWGT = 20
NC, NS = 2, 16
NW = NC * NS
NPAD = 10240
RPT = NPAD // NW
RPS = NPAD // NS
EPT = EE // NW
ECH = 200
NCHUNK = EPT // ECH

_MESH = plsc.VectorSubcoreMesh(
    core_axis_name="c", subcore_axis_name="s", num_cores=NC, num_subcores=NS)


def _embed_body(table, idx, out, idx_v, rows_v, sem):
  wid = lax.axis_index("s") * NC + lax.axis_index("c")
  base = wid * RPT
  pltpu.sync_copy(idx.at[pl.ds(base, RPT)], idx_v)
  pltpu.async_copy(table.at[idx_v], rows_v, sem).wait()
  pltpu.sync_copy(rows_v, out.at[pl.ds(base, RPT)])


_embed_gather = pl.kernel(
    _embed_body,
    out_type=jax.ShapeDtypeStruct((NPAD, DD), jnp.float32),
    mesh=_MESH,
    scratch_types=[
        pltpu.VMEM((RPT,), jnp.int32),
        pltpu.VMEM((RPT, DD), jnp.float32),
        pltpu.SemaphoreType.DMA,
    ],
)


def _agg_body(tbl, src2, dst, zrows, part, acc_sh, src_v, dst_v, rows_v, sem):
  c = lax.axis_index("c")
  s = lax.axis_index("s")
  wid = s * NC + c
  rbase = s * RPS
  pltpu.sync_copy(zrows.at[pl.ds(rbase, RPS)], acc_sh.at[pl.ds(rbase, RPS)])
  plsc.subcore_barrier()

  ebase = wid * EPT

  def chunk(i, carry):
    b = ebase + i * ECH
    pltpu.sync_copy(src2.at[pl.ds(b, ECH)], src_v)
    pltpu.sync_copy(dst.at[pl.ds(b, ECH)], dst_v)
    pltpu.async_copy(tbl.at[src_v], rows_v, sem).wait()
    pltpu.sync_copy(rows_v, acc_sh.at[dst_v], add=True)
    return carry

  lax.fori_loop(0, NCHUNK, chunk, 0)
  plsc.subcore_barrier()
  pltpu.sync_copy(acc_sh.at[pl.ds(rbase, RPS)], part.at[c, pl.ds(rbase, RPS)])


_edge_aggregate = pl.kernel(
    _agg_body,
    out_type=jax.ShapeDtypeStruct((NC, NPAD, DD), jnp.float32),
    mesh=_MESH,
    scratch_types=[
        pltpu.VMEM_SHARED((NPAD, DD), jnp.float32),
        pltpu.VMEM((ECH,), jnp.int32),
        pltpu.VMEM((ECH,), jnp.int32),
        pltpu.VMEM((ECH, DD), jnp.float32),
        pltpu.SemaphoreType.DMA,
    ],
)


def _prep_body(ei0_ref, ea_ref, src2_ref):
  src2_ref[...] = ea_ref[...] * NN + ei0_ref[...]


def _prep_src2(ei0, ea):
  return pl.pallas_call(
      _prep_body,
      out_shape=jax.ShapeDtypeStruct((EE // DD, DD), jnp.int32),
  )(ei0.reshape(EE // DD, DD), ea.reshape(EE // DD, DD)).reshape(EE)


_MB = 400


def _mscale_body(h_ref, w_ref, tbl_ref):
  mb = jnp.dot(h_ref[...], w_ref[...], preferred_element_type=jnp.float32)
  for w in range(NWGT):
    tbl_ref[w] = mb * jnp.float32(w)


def _mscale(h, w):
  return pl.pallas_call(
      _mscale_body,
      grid=(NN // _MB,),
      in_specs=[
          pl.BlockSpec((_MB, DD), lambda i: (i, 0)),
          pl.BlockSpec((DD, DD), lambda i: (0, 0)),
      ],
      out_specs=pl.BlockSpec((NWGT, _MB, DD), lambda i: (0, i, 0)),
      out_shape=jax.ShapeDtypeStruct((NWGT, NN, DD), jnp.float32),
  )(h, w).reshape(NWGT * NN, DD)


_GB = 1024


def _gru_body(part_ref, h_ref, wih_ref, whh_ref, bih_ref, bhh_ref, out_ref):
  agg = part_ref[0] + part_ref[1]
  h = h_ref[...]
  gi = lax.dot_general(agg, wih_ref[...], (((1,), (1,)), ((), ())),
                       preferred_element_type=jnp.float32) + bih_ref[...]
  gh = lax.dot_general(h, whh_ref[...], (((1,), (1,)), ((), ())),
                       preferred_element_type=jnp.float32) + bhh_ref[...]
  i_r, i_z, i_n = gi[:, :DD], gi[:, DD:2 * DD], gi[:, 2 * DD:]
  h_r, h_z, h_n = gh[:, :DD], gh[:, DD:2 * DD], gh[:, 2 * DD:]
  r = jax.nn.sigmoid(i_r + h_r)
  z = jax.nn.sigmoid(i_z + h_z)
  n = jnp.tanh(i_n + r * h_n)
  out_ref[...] = (1.0 - z) * n + z * h


def _gru(part, h, w_ih, w_hh, b_ih, b_hh):
  return pl.pallas_call(
      _gru_body,
      grid=(NPAD // _GB,),
      in_specs=[
          pl.BlockSpec((NC, _GB, DD), lambda i: (0, i, 0)),
          pl.BlockSpec((_GB, DD), lambda i: (i, 0)),
          pl.BlockSpec((3 * DD, DD), lambda i: (0, 0)),
          pl.BlockSpec((3 * DD, DD), lambda i: (0, 0)),
          pl.BlockSpec((1, 3 * DD), lambda i: (0, 0)),
          pl.BlockSpec((1, 3 * DD), lambda i: (0, 0)),
      ],
      out_specs=pl.BlockSpec((_GB, DD), lambda i: (i, 0)),
      out_shape=jax.ShapeDtypeStruct((NPAD, DD), jnp.float32),
  )(part, h, w_ih, w_hh, b_ih.reshape(1, 3 * DD), b_hh.reshape(1, 3 * DD))


def _pool_body(h_ref, gw_ref, gb_ref, lw_ref, lb_ref, y_ref):
  h = h_ref[...]
  scores = jnp.sum(h * gw_ref[...], axis=1, keepdims=True) + gb_ref[0, 0]
  gate = jax.nn.sigmoid(scores)
  rid = lax.broadcasted_iota(jnp.int32, (NPAD, 1), 0)
  valid = rid < NN
  gate = jnp.where(valid, gate, -jnp.inf)
  gate = gate - jnp.max(gate, axis=0, keepdims=True)
  e = jnp.exp(gate)
  e = jnp.where(valid, e, 0.0)
  g = e / jnp.sum(e, axis=0, keepdims=True)
  hg = jnp.sum(g * h, axis=0, keepdims=True)
  y_ref[...] = lax.dot_general(hg, lw_ref[...], (((1,), (1,)), ((), ())),
                               preferred_element_type=jnp.float32) + lb_ref[...]


def _pool(h, gate_w, gate_b, label_w, label_b):
  return pl.pallas_call(
      _pool_body,
      out_shape=jax.ShapeDtypeStruct((1, 2), jnp.float32),
  )(h, gate_w, gate_b.reshape(1, 1), label_w, label_b.reshape(1, 2))


@jax.jit
def kernel(x, edge_index, edge_attr, embed_table, edge_embed_table,
           ggnn_weight, gru_w_ih, gru_w_hh, gru_b_ih, gru_b_hh,
           gate_w, gate_b, label_w, label_b):
  del edge_embed_table

  xpad = jnp.concatenate(
      [x[:, 0].astype(jnp.int32),
       jnp.zeros((NPAD - NN,), jnp.int32)])
  h = _embed_gather(embed_table, xpad)

  src2 = _prep_src2(edge_index[0].astype(jnp.int32),
                    edge_attr[:, 0].astype(jnp.int32))
  dst = edge_index[1].astype(jnp.int32)
  zrows = jnp.zeros((NPAD, DD), jnp.float32)

  for l in range(LL):
    tbl = _mscale(h, ggnn_weight[l])
    part = _edge_aggregate(tbl, src2, dst, zrows)
    h = _gru(part, h, gru_w_ih, gru_w_hh, gru_b_ih, gru_b_hh)

  return _pool(h, gate_w, gate_b, label_w, label_b)


# speedup vs baseline: 8.3418x; 1.3084x over previous
import functools

import jax
import jax.numpy as jnp
from jax import lax
from jax.experimental import pallas as pl
from jax.experimental.pallas import tpu as pltpu
from jax.experimental.pallas import tpu_sc as plsc

NN = 10000
EE = 320000
DD = 128
LL = 4
NWGT = 20
NC, NS = 2, 16
NW = NC * NS
NPAD = 10240
RPT = NPAD // NW
RPS = NPAD // NS
EPT = EE // NW
ECH = 80
NCH = EPT // ECH
NSLOT = 4
NGRP = (NCH - 1) // NSLOT

_MESH = plsc.VectorSubcoreMesh(
    core_axis_name="c", subcore_axis_name="s", num_cores=NC, num_subcores=NS)


def _embed_body(table, idx, out, idx_v, rows_v, sem):
  wid = lax.axis_index("s") * NC + lax.axis_index("c")
  base = wid * RPT
  pltpu.sync_copy(idx.at[pl.ds(base, RPT)], idx_v)
  pltpu.async_copy(table.at[idx_v], rows_v, sem).wait()
  pltpu.sync_copy(rows_v, out.at[pl.ds(base, RPT)])


_embed_gather = pl.kernel(
    _embed_body,
    out_type=jax.ShapeDtypeStruct((NPAD, DD), jnp.float32),
    mesh=_MESH,
    scratch_types=[
        pltpu.VMEM((RPT,), jnp.int32),
        pltpu.VMEM((RPT, DD), jnp.float32),
        pltpu.SemaphoreType.DMA,
    ],
)


def _agg_body(tbl, src2, dst, zrows, part, acc_sh, sidx, didx, rows,
              isem, gsem, ssem):
  c = lax.axis_index("c")
  s = lax.axis_index("s")
  wid = s * NC + c
  rbase = s * RPS
  pltpu.sync_copy(zrows.at[pl.ds(rbase, RPS)], acc_sh.at[pl.ds(rbase, RPS)])
  plsc.subcore_barrier()

  ebase = wid * EPT

  def idx_descs(b, ci):
    eb = ebase + ci * ECH
    return (pltpu.make_async_copy(src2.at[pl.ds(eb, ECH)], sidx.at[b],
                                  isem.at[b]),
            pltpu.make_async_copy(dst.at[pl.ds(eb, ECH)], didx.at[b],
                                  isem.at[b]))

  def gat_desc(b):
    return pltpu.make_async_copy(tbl.at[sidx.at[b]], rows.at[b], gsem.at[b])

  def scat_desc(b):
    return pltpu.make_async_copy(rows.at[b], acc_sh.at[didx.at[b]],
                                 ssem.at[b])

  def group_body(j, carry):
    idescs = []
    for b in range(NSLOT):
      @pl.when(j > 0)
      def _wait_prev(b=b):
        scat_desc(b).wait()
      d = idx_descs(b, j * NSLOT + b)
      d[0].start()
      d[1].start()
      idescs.append(d)
    gdescs = []
    for b in range(NSLOT):
      idescs[b][0].wait()
      idescs[b][1].wait()
      g = gat_desc(b)
      g.start()
      gdescs.append(g)
    for b in range(NSLOT):
      gdescs[b].wait()
      scat_desc(b).start(add=True)
    return carry

  lax.fori_loop(0, NGRP, group_body, 0)

  scat_desc(0).wait()
  d1, d2 = idx_descs(0, NCH - 1)
  d1.start()
  d2.start()
  d1.wait()
  d2.wait()
  g = gat_desc(0)
  g.start()
  g.wait()
  scat_desc(0).start(add=True)
  scat_desc(0).wait()
  for b in range(1, NSLOT):
    scat_desc(b).wait()

  plsc.subcore_barrier()
  pltpu.sync_copy(acc_sh.at[pl.ds(rbase, RPS)], part.at[c, pl.ds(rbase, RPS)])


_edge_aggregate = pl.kernel(
    _agg_body,
    out_type=jax.ShapeDtypeStruct((NC, NPAD, DD), jnp.float32),
    mesh=_MESH,
    scratch_types=[
        pltpu.VMEM_SHARED((NPAD, DD), jnp.float32),
        pltpu.VMEM((NSLOT, ECH), jnp.int32),
        pltpu.VMEM((NSLOT, ECH), jnp.int32),
        pltpu.VMEM((NSLOT, ECH, DD), jnp.float32),
        pltpu.SemaphoreType.DMA((NSLOT,)),
        pltpu.SemaphoreType.DMA((NSLOT,)),
        pltpu.SemaphoreType.DMA((NSLOT,)),
    ],
)


def _prep_body(ei0_ref, ea_ref, src2_ref):
  src2_ref[...] = ea_ref[...] * NN + ei0_ref[...]


def _prep_src2(ei0, ea):
  return pl.pallas_call(
      _prep_body,
      out_shape=jax.ShapeDtypeStruct((EE // DD, DD), jnp.int32),
  )(ei0.reshape(EE // DD, DD), ea.reshape(EE // DD, DD)).reshape(EE)


_MB = 400


def _mscale_body(h_ref, w_ref, tbl_ref):
  mb = jnp.dot(h_ref[...], w_ref[...], preferred_element_type=jnp.float32)
  for w in range(NWGT):
    tbl_ref[w] = mb * jnp.float32(w)


def _mscale(h, w):
  return pl.pallas_call(
      _mscale_body,
      grid=(NN // _MB,),
      in_specs=[
          pl.BlockSpec((_MB, DD), lambda i: (i, 0)),
          pl.BlockSpec((DD, DD), lambda i: (0, 0)),
      ],
      out_specs=pl.BlockSpec((NWGT, _MB, DD), lambda i: (0, i, 0)),
      out_shape=jax.ShapeDtypeStruct((NWGT, NN, DD), jnp.float32),
  )(h, w).reshape(NWGT * NN, DD)


_GB = 1024


def _gru_body(part_ref, h_ref, wih_ref, whh_ref, bih_ref, bhh_ref, out_ref):
  agg = part_ref[0] + part_ref[1]
  h = h_ref[...]
  gi = lax.dot_general(agg, wih_ref[...], (((1,), (1,)), ((), ())),
                       preferred_element_type=jnp.float32) + bih_ref[...]
  gh = lax.dot_general(h, whh_ref[...], (((1,), (1,)), ((), ())),
                       preferred_element_type=jnp.float32) + bhh_ref[...]
  i_r, i_z, i_n = gi[:, :DD], gi[:, DD:2 * DD], gi[:, 2 * DD:]
  h_r, h_z, h_n = gh[:, :DD], gh[:, DD:2 * DD], gh[:, 2 * DD:]
  r = jax.nn.sigmoid(i_r + h_r)
  z = jax.nn.sigmoid(i_z + h_z)
  n = jnp.tanh(i_n + r * h_n)
  out_ref[...] = (1.0 - z) * n + z * h


def _gru(part, h, w_ih, w_hh, b_ih, b_hh):
  return pl.pallas_call(
      _gru_body,
      grid=(NPAD // _GB,),
      in_specs=[
          pl.BlockSpec((NC, _GB, DD), lambda i: (0, i, 0)),
          pl.BlockSpec((_GB, DD), lambda i: (i, 0)),
          pl.BlockSpec((3 * DD, DD), lambda i: (0, 0)),
          pl.BlockSpec((3 * DD, DD), lambda i: (0, 0)),
          pl.BlockSpec((1, 3 * DD), lambda i: (0, 0)),
          pl.BlockSpec((1, 3 * DD), lambda i: (0, 0)),
      ],
      out_specs=pl.BlockSpec((_GB, DD), lambda i: (i, 0)),
      out_shape=jax.ShapeDtypeStruct((NPAD, DD), jnp.float32),
  )(part, h, w_ih, w_hh, b_ih.reshape(1, 3 * DD), b_hh.reshape(1, 3 * DD))


def _pool_body(h_ref, gw_ref, gb_ref, lw_ref, lb_ref, y_ref):
  h = h_ref[...]
  scores = jnp.sum(h * gw_ref[...], axis=1, keepdims=True) + gb_ref[0, 0]
  gate = jax.nn.sigmoid(scores)
  rid = lax.broadcasted_iota(jnp.int32, (NPAD, 1), 0)
  valid = rid < NN
  gate = jnp.where(valid, gate, -jnp.inf)
  gate = gate - jnp.max(gate, axis=0, keepdims=True)
  e = jnp.exp(gate)
  e = jnp.where(valid, e, 0.0)
  g = e / jnp.sum(e, axis=0, keepdims=True)
  hg = jnp.sum(g * h, axis=0, keepdims=True)
  y_ref[...] = lax.dot_general(hg, lw_ref[...], (((1,), (1,)), ((), ())),
                               preferred_element_type=jnp.float32) + lb_ref[...]


def _pool(h, gate_w, gate_b, label_w, label_b):
  return pl.pallas_call(
      _pool_body,
      out_shape=jax.ShapeDtypeStruct((1, 2), jnp.float32),
  )(h, gate_w, gate_b.reshape(1, 1), label_w, label_b.reshape(1, 2))


@jax.jit
def kernel(x, edge_index, edge_attr, embed_table, edge_embed_table,
           ggnn_weight, gru_w_ih, gru_w_hh, gru_b_ih, gru_b_hh,
           gate_w, gate_b, label_w, label_b):
  del edge_embed_table

  xpad = jnp.concatenate(
      [x[:, 0].astype(jnp.int32),
       jnp.zeros((NPAD - NN,), jnp.int32)])
  h = _embed_gather(embed_table, xpad)

  src2 = _prep_src2(edge_index[0].astype(jnp.int32),
                    edge_attr[:, 0].astype(jnp.int32))
  dst = edge_index[1].astype(jnp.int32)
  zrows = jnp.zeros((NPAD, DD), jnp.float32)

  for l in range(LL):
    tbl = _mscale(h, ggnn_weight[l])
    part = _edge_aggregate(tbl, src2, dst, zrows)
    h = _gru(part, h, gru_w_ih, gru_w_hh, gru_b_ih, gru_b_hh)

  return _pool(h, gate_w, gate_b, label_w, label_b)


# speedup vs baseline: 8.5133x; 1.0206x over previous
import functools

import jax
import jax.numpy as jnp
from jax import lax
from jax.experimental import pallas as pl
from jax.experimental.pallas import tpu as pltpu
from jax.experimental.pallas import tpu_sc as plsc

NN = 10000
EE = 320000
DD = 128
LL = 4
NWGT = 20
NC, NS = 2, 16
NW = NC * NS
NPAD = 10240
RPT = NPAD // NW
RPS = NPAD // NS
EPT = EE // NW
ECH = 40
NCH = EPT // ECH
NSLOT = 8
NGRP = NCH // NSLOT
NTAIL = NCH - NGRP * NSLOT

_MESH = plsc.VectorSubcoreMesh(
    core_axis_name="c", subcore_axis_name="s", num_cores=NC, num_subcores=NS)


def _embed_body(table, idx, out, idx_v, rows_v, sem):
  wid = lax.axis_index("s") * NC + lax.axis_index("c")
  base = wid * RPT
  pltpu.sync_copy(idx.at[pl.ds(base, RPT)], idx_v)
  pltpu.async_copy(table.at[idx_v], rows_v, sem).wait()
  pltpu.sync_copy(rows_v, out.at[pl.ds(base, RPT)])


_embed_gather = pl.kernel(
    _embed_body,
    out_type=jax.ShapeDtypeStruct((NPAD, DD), jnp.float32),
    mesh=_MESH,
    scratch_types=[
        pltpu.VMEM((RPT,), jnp.int32),
        pltpu.VMEM((RPT, DD), jnp.float32),
        pltpu.SemaphoreType.DMA,
    ],
)


def _agg_body(tbl, src2, dst, zrows, part, acc_sh, sidx, didx, rows,
              isem, gsem, ssem):
  c = lax.axis_index("c")
  s = lax.axis_index("s")
  wid = s * NC + c
  rbase = s * RPS
  pltpu.sync_copy(zrows.at[pl.ds(rbase, RPS)], acc_sh.at[pl.ds(rbase, RPS)])
  plsc.subcore_barrier()

  ebase = wid * EPT

  def idx_descs(b, ci):
    eb = ebase + ci * ECH
    return (pltpu.make_async_copy(src2.at[pl.ds(eb, ECH)], sidx.at[b],
                                  isem.at[b]),
            pltpu.make_async_copy(dst.at[pl.ds(eb, ECH)], didx.at[b],
                                  isem.at[b]))

  def gat_desc(b):
    return pltpu.make_async_copy(tbl.at[sidx.at[b]], rows.at[b], gsem.at[b])

  def scat_desc(b):
    return pltpu.make_async_copy(rows.at[b], acc_sh.at[didx.at[b]],
                                 ssem.at[b])

  def group_body(j, carry):
    idescs = []
    for b in range(NSLOT):
      @pl.when(j > 0)
      def _wait_prev(b=b):
        scat_desc(b).wait()
      d = idx_descs(b, j * NSLOT + b)
      d[0].start()
      d[1].start()
      idescs.append(d)
    gdescs = []
    for b in range(NSLOT):
      idescs[b][0].wait()
      idescs[b][1].wait()
      g = gat_desc(b)
      g.start()
      gdescs.append(g)
    for b in range(NSLOT):
      gdescs[b].wait()
      scat_desc(b).start(add=True)
    return carry

  lax.fori_loop(0, NGRP, group_body, 0)

  for k in range(NTAIL):
    scat_desc(k).wait()
    d1, d2 = idx_descs(k, NGRP * NSLOT + k)
    d1.start()
    d2.start()
    d1.wait()
    d2.wait()
    g = gat_desc(k)
    g.start()
    g.wait()
    scat_desc(k).start(add=True)
  for b in range(NSLOT):
    scat_desc(b).wait()

  plsc.subcore_barrier()
  pltpu.sync_copy(acc_sh.at[pl.ds(rbase, RPS)], part.at[c, pl.ds(rbase, RPS)])


_edge_aggregate = pl.kernel(
    _agg_body,
    out_type=jax.ShapeDtypeStruct((NC, NPAD, DD), jnp.float32),
    mesh=_MESH,
    scratch_types=[
        pltpu.VMEM_SHARED((NPAD, DD), jnp.float32),
        pltpu.VMEM((NSLOT, ECH), jnp.int32),
        pltpu.VMEM((NSLOT, ECH), jnp.int32),
        pltpu.VMEM((NSLOT, ECH, DD), jnp.float32),
        pltpu.SemaphoreType.DMA((NSLOT,)),
        pltpu.SemaphoreType.DMA((NSLOT,)),
        pltpu.SemaphoreType.DMA((NSLOT,)),
    ],
)


def _prep_body(ei0_ref, ea_ref, src2_ref):
  src2_ref[...] = ea_ref[...] * NN + ei0_ref[...]


def _prep_src2(ei0, ea):
  return pl.pallas_call(
      _prep_body,
      out_shape=jax.ShapeDtypeStruct((EE // DD, DD), jnp.int32),
  )(ei0.reshape(EE // DD, DD), ea.reshape(EE // DD, DD)).reshape(EE)


_MB = 400


def _mscale_body(h_ref, w_ref, tbl_ref):
  mb = jnp.dot(h_ref[...], w_ref[...], preferred_element_type=jnp.float32)
  for w in range(NWGT):
    tbl_ref[w] = mb * jnp.float32(w)


def _mscale(h, w):
  return pl.pallas_call(
      _mscale_body,
      grid=(NN // _MB,),
      in_specs=[
          pl.BlockSpec((_MB, DD), lambda i: (i, 0)),
          pl.BlockSpec((DD, DD), lambda i: (0, 0)),
      ],
      out_specs=pl.BlockSpec((NWGT, _MB, DD), lambda i: (0, i, 0)),
      out_shape=jax.ShapeDtypeStruct((NWGT, NN, DD), jnp.float32),
  )(h, w).reshape(NWGT * NN, DD)


_GB = 1024


def _gru_body(part_ref, h_ref, wih_ref, whh_ref, bih_ref, bhh_ref, out_ref):
  agg = part_ref[0] + part_ref[1]
  h = h_ref[...]
  gi = lax.dot_general(agg, wih_ref[...], (((1,), (1,)), ((), ())),
                       preferred_element_type=jnp.float32) + bih_ref[...]
  gh = lax.dot_general(h, whh_ref[...], (((1,), (1,)), ((), ())),
                       preferred_element_type=jnp.float32) + bhh_ref[...]
  i_r, i_z, i_n = gi[:, :DD], gi[:, DD:2 * DD], gi[:, 2 * DD:]
  h_r, h_z, h_n = gh[:, :DD], gh[:, DD:2 * DD], gh[:, 2 * DD:]
  r = jax.nn.sigmoid(i_r + h_r)
  z = jax.nn.sigmoid(i_z + h_z)
  n = jnp.tanh(i_n + r * h_n)
  out_ref[...] = (1.0 - z) * n + z * h


def _gru(part, h, w_ih, w_hh, b_ih, b_hh):
  return pl.pallas_call(
      _gru_body,
      grid=(NPAD // _GB,),
      in_specs=[
          pl.BlockSpec((NC, _GB, DD), lambda i: (0, i, 0)),
          pl.BlockSpec((_GB, DD), lambda i: (i, 0)),
          pl.BlockSpec((3 * DD, DD), lambda i: (0, 0)),
          pl.BlockSpec((3 * DD, DD), lambda i: (0, 0)),
          pl.BlockSpec((1, 3 * DD), lambda i: (0, 0)),
          pl.BlockSpec((1, 3 * DD), lambda i: (0, 0)),
      ],
      out_specs=pl.BlockSpec((_GB, DD), lambda i: (i, 0)),
      out_shape=jax.ShapeDtypeStruct((NPAD, DD), jnp.float32),
  )(part, h, w_ih, w_hh, b_ih.reshape(1, 3 * DD), b_hh.reshape(1, 3 * DD))


def _pool_body(h_ref, gw_ref, gb_ref, lw_ref, lb_ref, y_ref):
  h = h_ref[...]
  scores = jnp.sum(h * gw_ref[...], axis=1, keepdims=True) + gb_ref[0, 0]
  gate = jax.nn.sigmoid(scores)
  rid = lax.broadcasted_iota(jnp.int32, (NPAD, 1), 0)
  valid = rid < NN
  gate = jnp.where(valid, gate, -jnp.inf)
  gate = gate - jnp.max(gate, axis=0, keepdims=True)
  e = jnp.exp(gate)
  e = jnp.where(valid, e, 0.0)
  g = e / jnp.sum(e, axis=0, keepdims=True)
  hg = jnp.sum(g * h, axis=0, keepdims=True)
  y_ref[...] = lax.dot_general(hg, lw_ref[...], (((1,), (1,)), ((), ())),
                               preferred_element_type=jnp.float32) + lb_ref[...]


def _pool(h, gate_w, gate_b, label_w, label_b):
  return pl.pallas_call(
      _pool_body,
      out_shape=jax.ShapeDtypeStruct((1, 2), jnp.float32),
  )(h, gate_w, gate_b.reshape(1, 1), label_w, label_b.reshape(1, 2))


@jax.jit
def kernel(x, edge_index, edge_attr, embed_table, edge_embed_table,
           ggnn_weight, gru_w_ih, gru_w_hh, gru_b_ih, gru_b_hh,
           gate_w, gate_b, label_w, label_b):
  del edge_embed_table

  xpad = jnp.concatenate(
      [x[:, 0].astype(jnp.int32),
       jnp.zeros((NPAD - NN,), jnp.int32)])
  h = _embed_gather(embed_table, xpad)

  src2 = _prep_src2(edge_index[0].astype(jnp.int32),
                    edge_attr[:, 0].astype(jnp.int32))
  dst = edge_index[1].astype(jnp.int32)
  zrows = jnp.zeros((NPAD, DD), jnp.float32)

  for l in range(LL):
    tbl = _mscale(h, ggnn_weight[l])
    part = _edge_aggregate(tbl, src2, dst, zrows)
    h = _gru(part, h, gru_w_ih, gru_w_hh, gru_b_ih, gru_b_hh)

  return _pool(h, gate_w, gate_b, label_w, label_b)


# speedup vs baseline: 8.8153x; 1.0355x over previous
import functools

import jax
import jax.numpy as jnp
from jax import lax
from jax.experimental import pallas as pl
from jax.experimental.pallas import tpu as pltpu
from jax.experimental.pallas import tpu_sc as plsc

NN = 10000
EE = 320000
DD = 128
LL = 4
NWGT = 20
NC, NS = 2, 16
NW = NC * NS
NPAD = 10240
RPT = NPAD // NW
RPS = NPAD // NS
EPT = EE // NW
ECH = 40
NCH = EPT // ECH
NSLOT = 8
NGRP = NCH // NSLOT
NTAIL = NCH - NGRP * NSLOT

_MESH = plsc.VectorSubcoreMesh(
    core_axis_name="c", subcore_axis_name="s", num_cores=NC, num_subcores=NS)


def _embed_body(table, idx, out, idx_v, rows_v, sem):
  wid = lax.axis_index("s") * NC + lax.axis_index("c")
  base = wid * RPT
  pltpu.sync_copy(idx.at[pl.ds(base, RPT)], idx_v)
  pltpu.async_copy(table.at[idx_v], rows_v, sem).wait()
  pltpu.sync_copy(rows_v, out.at[pl.ds(base, RPT)])


_embed_gather = pl.kernel(
    _embed_body,
    out_type=jax.ShapeDtypeStruct((NPAD, DD), jnp.float32),
    mesh=_MESH,
    scratch_types=[
        pltpu.VMEM((RPT,), jnp.int32),
        pltpu.VMEM((RPT, DD), jnp.float32),
        pltpu.SemaphoreType.DMA,
    ],
)


def _agg_body(tbl, src2, dst, zrows, part, acc_sh, sidx, didx, rows,
              isem, gsem, ssem):
  c = lax.axis_index("c")
  s = lax.axis_index("s")
  wid = s * NC + c
  rbase = s * RPS
  pltpu.sync_copy(zrows.at[pl.ds(rbase, RPS)], acc_sh.at[pl.ds(rbase, RPS)])
  plsc.subcore_barrier()

  ebase = wid * EPT

  def idx_descs(b, ci):
    eb = ebase + ci * ECH
    return (pltpu.make_async_copy(src2.at[pl.ds(eb, ECH)], sidx.at[b],
                                  isem.at[b]),
            pltpu.make_async_copy(dst.at[pl.ds(eb, ECH)], didx.at[b],
                                  isem.at[b]))

  def gat_desc(b):
    return pltpu.make_async_copy(tbl.at[sidx.at[b]], rows.at[b], gsem.at[b])

  def scat_desc(b):
    return pltpu.make_async_copy(rows.at[b], acc_sh.at[didx.at[b]],
                                 ssem.at[b])

  def group_body(j, carry):
    idescs = []
    for b in range(NSLOT):
      @pl.when(j > 0)
      def _wait_prev(b=b):
        scat_desc(b).wait()
      d = idx_descs(b, j * NSLOT + b)
      d[0].start()
      d[1].start()
      idescs.append(d)
    gdescs = []
    for b in range(NSLOT):
      idescs[b][0].wait()
      idescs[b][1].wait()
      g = gat_desc(b)
      g.start()
      gdescs.append(g)
    for b in range(NSLOT):
      gdescs[b].wait()
      scat_desc(b).start(add=True)
    return carry

  lax.fori_loop(0, NGRP, group_body, 0)

  for k in range(NTAIL):
    scat_desc(k).wait()
    d1, d2 = idx_descs(k, NGRP * NSLOT + k)
    d1.start()
    d2.start()
    d1.wait()
    d2.wait()
    g = gat_desc(k)
    g.start()
    g.wait()
    scat_desc(k).start(add=True)
  for b in range(NSLOT):
    scat_desc(b).wait()

  plsc.subcore_barrier()
  pltpu.sync_copy(acc_sh.at[pl.ds(rbase, RPS)], part.at[c, pl.ds(rbase, RPS)])


_edge_aggregate = pl.kernel(
    _agg_body,
    out_type=jax.ShapeDtypeStruct((NC, NPAD, DD), jnp.float32),
    mesh=_MESH,
    scratch_types=[
        pltpu.VMEM_SHARED((NPAD, DD), jnp.float32),
        pltpu.VMEM((NSLOT, ECH), jnp.int32),
        pltpu.VMEM((NSLOT, ECH), jnp.int32),
        pltpu.VMEM((NSLOT, ECH, DD), jnp.float32),
        pltpu.SemaphoreType.DMA((NSLOT,)),
        pltpu.SemaphoreType.DMA((NSLOT,)),
        pltpu.SemaphoreType.DMA((NSLOT,)),
    ],
)


def _prep_body(ei0_ref, ea_ref, src2_ref):
  src2_ref[...] = ea_ref[...] * NPAD + ei0_ref[...]


def _prep_src2(ei0, ea):
  return pl.pallas_call(
      _prep_body,
      out_shape=jax.ShapeDtypeStruct((EE // DD, DD), jnp.int32),
  )(ei0.reshape(EE // DD, DD), ea.reshape(EE // DD, DD)).reshape(EE)


_MB = 512


def _mscale_body(h_ref, w_ref, tbl_ref):
  mb = jnp.dot(h_ref[...], w_ref[...], preferred_element_type=jnp.float32)
  for w in range(NWGT):
    tbl_ref[w] = mb * jnp.float32(w)


def _mscale(h, w):
  return pl.pallas_call(
      _mscale_body,
      grid=(NPAD // _MB,),
      in_specs=[
          pl.BlockSpec((_MB, DD), lambda i: (i, 0)),
          pl.BlockSpec((DD, DD), lambda i: (0, 0)),
      ],
      out_specs=pl.BlockSpec((NWGT, _MB, DD), lambda i: (0, i, 0)),
      out_shape=jax.ShapeDtypeStruct((NWGT, NPAD, DD), jnp.float32),
  )(h, w).reshape(NWGT * NPAD, DD)


_GB = 1024


def _gru_body(part_ref, h_ref, wih_ref, whh_ref, bih_ref, bhh_ref, out_ref):
  agg = part_ref[0] + part_ref[1]
  h = h_ref[...]
  gi = lax.dot_general(agg, wih_ref[...], (((1,), (1,)), ((), ())),
                       preferred_element_type=jnp.float32) + bih_ref[...]
  gh = lax.dot_general(h, whh_ref[...], (((1,), (1,)), ((), ())),
                       preferred_element_type=jnp.float32) + bhh_ref[...]
  i_r, i_z, i_n = gi[:, :DD], gi[:, DD:2 * DD], gi[:, 2 * DD:]
  h_r, h_z, h_n = gh[:, :DD], gh[:, DD:2 * DD], gh[:, 2 * DD:]
  r = jax.nn.sigmoid(i_r + h_r)
  z = jax.nn.sigmoid(i_z + h_z)
  n = jnp.tanh(i_n + r * h_n)
  out_ref[...] = (1.0 - z) * n + z * h


def _gru(part, h, w_ih, w_hh, b_ih, b_hh):
  return pl.pallas_call(
      _gru_body,
      grid=(NPAD // _GB,),
      in_specs=[
          pl.BlockSpec((NC, _GB, DD), lambda i: (0, i, 0)),
          pl.BlockSpec((_GB, DD), lambda i: (i, 0)),
          pl.BlockSpec((3 * DD, DD), lambda i: (0, 0)),
          pl.BlockSpec((3 * DD, DD), lambda i: (0, 0)),
          pl.BlockSpec((1, 3 * DD), lambda i: (0, 0)),
          pl.BlockSpec((1, 3 * DD), lambda i: (0, 0)),
      ],
      out_specs=pl.BlockSpec((_GB, DD), lambda i: (i, 0)),
      out_shape=jax.ShapeDtypeStruct((NPAD, DD), jnp.float32),
  )(part, h, w_ih, w_hh, b_ih.reshape(1, 3 * DD), b_hh.reshape(1, 3 * DD))


def _gru_mscale_body(part_ref, h_ref, wih_ref, whh_ref, bih_ref, bhh_ref,
                     wn_ref, out_ref, tbl_ref):
  agg = part_ref[0] + part_ref[1]
  h = h_ref[...]
  gi = lax.dot_general(agg, wih_ref[...], (((1,), (1,)), ((), ())),
                       preferred_element_type=jnp.float32) + bih_ref[...]
  gh = lax.dot_general(h, whh_ref[...], (((1,), (1,)), ((), ())),
                       preferred_element_type=jnp.float32) + bhh_ref[...]
  i_r, i_z, i_n = gi[:, :DD], gi[:, DD:2 * DD], gi[:, 2 * DD:]
  h_r, h_z, h_n = gh[:, :DD], gh[:, DD:2 * DD], gh[:, 2 * DD:]
  r = jax.nn.sigmoid(i_r + h_r)
  z = jax.nn.sigmoid(i_z + h_z)
  n = jnp.tanh(i_n + r * h_n)
  hn = (1.0 - z) * n + z * h
  out_ref[...] = hn
  mb = jnp.dot(hn, wn_ref[...], preferred_element_type=jnp.float32)
  for w in range(NWGT):
    tbl_ref[w] = mb * jnp.float32(w)


_FB = 512


def _gru_mscale(part, h, w_ih, w_hh, b_ih, b_hh, w_next):
  hn, tbl = pl.pallas_call(
      _gru_mscale_body,
      grid=(NPAD // _FB,),
      in_specs=[
          pl.BlockSpec((NC, _FB, DD), lambda i: (0, i, 0)),
          pl.BlockSpec((_FB, DD), lambda i: (i, 0)),
          pl.BlockSpec((3 * DD, DD), lambda i: (0, 0)),
          pl.BlockSpec((3 * DD, DD), lambda i: (0, 0)),
          pl.BlockSpec((1, 3 * DD), lambda i: (0, 0)),
          pl.BlockSpec((1, 3 * DD), lambda i: (0, 0)),
          pl.BlockSpec((DD, DD), lambda i: (0, 0)),
      ],
      out_specs=[
          pl.BlockSpec((_FB, DD), lambda i: (i, 0)),
          pl.BlockSpec((NWGT, _FB, DD), lambda i: (0, i, 0)),
      ],
      out_shape=[
          jax.ShapeDtypeStruct((NPAD, DD), jnp.float32),
          jax.ShapeDtypeStruct((NWGT, NPAD, DD), jnp.float32),
      ],
  )(part, h, w_ih, w_hh, b_ih.reshape(1, 3 * DD), b_hh.reshape(1, 3 * DD),
    w_next)
  return hn, tbl.reshape(NWGT * NPAD, DD)


def _pool_body(h_ref, gw_ref, gb_ref, lw_ref, lb_ref, y_ref):
  h = h_ref[...]
  scores = jnp.sum(h * gw_ref[...], axis=1, keepdims=True) + gb_ref[0, 0]
  gate = jax.nn.sigmoid(scores)
  rid = lax.broadcasted_iota(jnp.int32, (NPAD, 1), 0)
  valid = rid < NN
  gate = jnp.where(valid, gate, -jnp.inf)
  gate = gate - jnp.max(gate, axis=0, keepdims=True)
  e = jnp.exp(gate)
  e = jnp.where(valid, e, 0.0)
  g = e / jnp.sum(e, axis=0, keepdims=True)
  hg = jnp.sum(g * h, axis=0, keepdims=True)
  y_ref[...] = lax.dot_general(hg, lw_ref[...], (((1,), (1,)), ((), ())),
                               preferred_element_type=jnp.float32) + lb_ref[...]


def _pool(h, gate_w, gate_b, label_w, label_b):
  return pl.pallas_call(
      _pool_body,
      out_shape=jax.ShapeDtypeStruct((1, 2), jnp.float32),
  )(h, gate_w, gate_b.reshape(1, 1), label_w, label_b.reshape(1, 2))


@jax.jit
def kernel(x, edge_index, edge_attr, embed_table, edge_embed_table,
           ggnn_weight, gru_w_ih, gru_w_hh, gru_b_ih, gru_b_hh,
           gate_w, gate_b, label_w, label_b):
  del edge_embed_table

  xpad = jnp.concatenate(
      [x[:, 0].astype(jnp.int32),
       jnp.zeros((NPAD - NN,), jnp.int32)])
  h = _embed_gather(embed_table, xpad)

  src2 = _prep_src2(edge_index[0].astype(jnp.int32),
                    edge_attr[:, 0].astype(jnp.int32))
  dst = edge_index[1].astype(jnp.int32)
  zrows = jnp.zeros((NPAD, DD), jnp.float32)

  tbl = _mscale(h, ggnn_weight[0])
  for l in range(LL):
    part = _edge_aggregate(tbl, src2, dst, zrows)
    if l < LL - 1:
      h, tbl = _gru_mscale(part, h, gru_w_ih, gru_w_hh, gru_b_ih,
                           gru_b_hh, ggnn_weight[l + 1])
    else:
      h = _gru(part, h, gru_w_ih, gru_w_hh, gru_b_ih, gru_b_hh)

  return _pool(h, gate_w, gate_b, label_w, label_b)
